# baseline (speedup 1.0000x reference)
import functools

import jax
import jax.numpy as jnp
from jax import lax
from jax.experimental import pallas as pl
from jax.experimental.pallas import tpu as pltpu
from jax.experimental.pallas import tpu_sc as plsc

B = 16384
D = 32
NC = 2
NS = 16
NW = NC * NS
BPW = B // NW
CH = 128
NCH = BPW // CH

_sc_mesh = plsc.VectorSubcoreMesh(core_axis_name="c", subcore_axis_name="s")


@functools.partial(
    pl.kernel,
    mesh=_sc_mesh,
    compiler_params=pltpu.CompilerParams(use_tc_tiling_on_sc=False),
    out_type=[jax.ShapeDtypeStruct((B, D), jnp.float32)] * 4,
    scratch_types=[
        pltpu.VMEM((NCH, CH), jnp.int32),
        pltpu.VMEM((NCH, CH), jnp.int32),
        pltpu.VMEM((BPW, D), jnp.float32),
        pltpu.VMEM((BPW, D), jnp.float32),
        pltpu.VMEM((BPW, D), jnp.float32),
        pltpu.VMEM((BPW, D), jnp.float32),
        pltpu.SemaphoreType.DMA,
    ],
)
def _sc_gather(uids, mids, gue, gme, mue, mme,
               gu_o, gm_o, mu_o, mm_o,
               uidx_v, midx_v, gu_v, gm_v, mu_v, mm_v, sem):
    wid = lax.axis_index("s") * NC + lax.axis_index("c")
    base = wid * BPW
    for j in range(NCH):
        pltpu.sync_copy(uids.at[pl.ds(base + j * CH, CH)], uidx_v.at[j])
        pltpu.sync_copy(mids.at[pl.ds(base + j * CH, CH)], midx_v.at[j])
    copies = []
    for j in range(NCH):
        row = pl.ds(j * CH, CH)
        copies.append(pltpu.async_copy(gue.at[uidx_v.at[j]], gu_v.at[row], sem))
        copies.append(pltpu.async_copy(gme.at[midx_v.at[j]], gm_v.at[row], sem))
        copies.append(pltpu.async_copy(mue.at[uidx_v.at[j]], mu_v.at[row], sem))
        copies.append(pltpu.async_copy(mme.at[midx_v.at[j]], mm_v.at[row], sem))
    for c in copies:
        c.wait()
    out_rows = pl.ds(base, BPW)
    pltpu.sync_copy(gu_v, gu_o.at[out_rows])
    pltpu.sync_copy(gm_v, gm_o.at[out_rows])
    pltpu.sync_copy(mu_v, mu_o.at[out_rows])
    pltpu.sync_copy(mm_v, mm_o.at[out_rows])


BLK = 2048


def _tc_body(gu, gm, mu, mm, w1, b1, w2, b2, wt, bo, out_ref):
    gmf = gu[...] * gm[...]
    x = jnp.concatenate([mu[...], mm[...]], axis=1)
    h = jnp.maximum(jnp.dot(x, w1[...], preferred_element_type=jnp.float32) + b1[...], 0.0)
    h = jnp.maximum(jnp.dot(h, w2[...], preferred_element_type=jnp.float32) + b2[...], 0.0)
    cat = jnp.concatenate([gmf, h], axis=1)
    logit = jnp.sum(cat * wt[...], axis=1) + bo[...]
    out_ref[...] = 1.0 / (1.0 + jnp.exp(-logit))


def _tc_mlp(gu, gm, mu, mm, W1, b1, W2, b2, wt, bout):
    grid = B // BLK
    blk2 = lambda shape: pl.BlockSpec(shape, lambda i: (0, 0))
    blk1 = lambda shape: pl.BlockSpec(shape, lambda i: (0,))
    row_blk = pl.BlockSpec((BLK, D), lambda i: (i, 0))
    return pl.pallas_call(
        _tc_body,
        grid=(grid,),
        in_specs=[
            row_blk, row_blk, row_blk, row_blk,
            blk2(W1.shape), blk1(b1.shape),
            blk2(W2.shape), blk1(b2.shape),
            blk1(wt.shape), blk1(bout.shape),
        ],
        out_specs=pl.BlockSpec((BLK,), lambda i: (i,)),
        out_shape=jax.ShapeDtypeStruct((B,), jnp.float32),
    )(gu, gm, mu, mm, W1, b1, W2, b2, wt, bout)


def kernel(user_ids, movie_ids, gmf_user_emb, gmf_movie_emb,
           mlp_user_emb, mlp_movie_emb, W1, b1, W2, b2, Wout, bout):
    gu, gm, mu, mm = _sc_gather(user_ids, movie_ids, gmf_user_emb,
                                gmf_movie_emb, mlp_user_emb, mlp_movie_emb)
    wt = Wout[:, 0]
    return _tc_mlp(gu, gm, mu, mm, W1, b1, W2, b2, wt, bout)
